# baseline (speedup 1.0000x reference)
import dataclasses
import functools

import jax
import jax.numpy as jnp
from jax import lax
from jax.experimental import pallas as pl
from jax.experimental.pallas import tpu as pltpu
from jax.experimental.pallas import tpu_sc as plsc

_N = 10000
_D = 128
_E = 320000
_NC = 2
_NS = 16
_NW = _NC * _NS
_EPW = _E // _NW
_WIN = 80
_EPP = 10080
_PAD = _EPP - _EPW
_NPH = 3
_EPH = _EPP // _NPH
_WPH = _EPH // _WIN
_NBUF = 3
_HWIN = _WIN // 2

_PERM = []
for _j in range(0, _D, 32):
    for _t in range(16):
        _PERM.append(_j + _t)
        _PERM.append(_j + 16 + _t)
_CHUNK = 200
_NCHUNK = _N // _CHUNK
_NZCHUNK = _N // _HWIN
_LANES = 16

_BM = 400


def _pack_i32(h):
    return lax.bitcast_convert_type(
        h.reshape(h.shape[0], _D // 2, 2), jnp.int32)


def _mm_body(x_ref, w_ref, o_ref):
    o_ref[...] = lax.dot_general(
        x_ref[...], w_ref[...], (((1,), (1,)), ((), ())),
        precision=lax.Precision.HIGHEST,
        preferred_element_type=jnp.float32).astype(jnp.bfloat16)


def _tc_matmul(x, w):
    return pl.pallas_call(
        _mm_body,
        grid=(_N // _BM,),
        in_specs=[pl.BlockSpec((_BM, _D), lambda i: (i, 0)),
                  pl.BlockSpec((_D, _D), lambda i: (0, 0))],
        out_specs=pl.BlockSpec((_BM, _D), lambda i: (i, 0)),
        out_shape=jax.ShapeDtypeStruct((_N, _D), jnp.bfloat16),
    )(x, w)


def _fuse_body(p_ref, w_ref, o_ref):
    h = jnp.maximum(p_ref[0] + p_ref[1], 0.0)
    o_ref[...] = lax.dot_general(
        h, w_ref[...], (((1,), (1,)), ((), ())),
        precision=lax.Precision.HIGHEST,
        preferred_element_type=jnp.float32).astype(jnp.bfloat16)


def _tc_relu_add_matmul(p, w):
    return pl.pallas_call(
        _fuse_body,
        grid=(_N // _BM,),
        in_specs=[pl.BlockSpec((_NC, _BM, _D), lambda i: (0, i, 0)),
                  pl.BlockSpec((_D, _D), lambda i: (0, 0))],
        out_specs=pl.BlockSpec((_BM, _D), lambda i: (i, 0)),
        out_shape=jax.ShapeDtypeStruct((_N, _D), jnp.bfloat16),
    )(p, w)


def _add_body(q_ref, o_ref):
    o_ref[...] = q_ref[0] + q_ref[1]


def _tc_add(q):
    return pl.pallas_call(
        _add_body,
        grid=(_N // _BM,),
        in_specs=[pl.BlockSpec((_NC, _BM, _D), lambda i: (0, i, 0))],
        out_specs=pl.BlockSpec((_BM, _D), lambda i: (i, 0)),
        out_shape=jax.ShapeDtypeStruct((_N, _D), jnp.float32),
    )(q)


def _sc_spmm(h, row3, col2, ev2):
    mesh = plsc.VectorSubcoreMesh(core_axis_name="c", subcore_axis_name="s")
    cp = pltpu.CompilerParams()
    if "needs_layout_passes" in pltpu.CompilerParams.__dataclass_fields__:
        cp = dataclasses.replace(cp, needs_layout_passes=False)
    if "use_tc_tiling_on_sc" in pltpu.CompilerParams.__dataclass_fields__:
        cp = dataclasses.replace(cp, use_tc_tiling_on_sc=False)

    @functools.partial(
        pl.kernel,
        out_type=jax.ShapeDtypeStruct((_NC, _N, _D), jnp.float32),
        mesh=mesh,
        compiler_params=cp,
        scratch_types=[
            pltpu.VMEM((_EPH,), jnp.int32),
            pltpu.VMEM((2 * _WPH, _HWIN), jnp.int32),
            pltpu.VMEM((_EPH,), jnp.float32),
            pltpu.VMEM((_NBUF, _WIN, _D // 2), jnp.int32),
            pltpu.VMEM((_NBUF, _HWIN, _D), jnp.float32),
            pltpu.VMEM_SHARED((_N, _D), jnp.float32),
            pltpu.SemaphoreType.DMA,
            pltpu.SemaphoreType.DMA,
            pltpu.SemaphoreType.DMA,
            pltpu.SemaphoreType.DMA,
            pltpu.SemaphoreType.DMA,
            pltpu.SemaphoreType.DMA,
        ],
    )
    def spmm(h_hbm, row_hbm, col_hbm, ev_hbm, out_hbm,
             col_v, row_v, ev_v, gbuf, sbuf, acc,
             gs0, gs1, gs2, ss0, ss1, ss2):
        c = lax.axis_index("c")
        s = lax.axis_index("s")
        wid = s * _NC + c
        gsem = (gs0, gs1, gs2)
        ssem = (ss0, ss1, ss2)

        zb = sbuf.at[0]

        @pl.loop(0, _HWIN)
        def _zero_stage(i):
            for j in range(0, _D, _LANES):
                zb[i, pl.ds(j, _LANES)] = jnp.zeros((_LANES,), jnp.float32)

        @pl.loop(0, pl.cdiv(_NZCHUNK, _NS))
        def _zero_acc(k):
            chunk = s + k * _NS

            @pl.when(chunk < _NZCHUNK)
            def _():
                pltpu.sync_copy(zb, acc.at[pl.ds(chunk * _HWIN, _HWIN)])

        plsc.subcore_barrier()

        def start_gather(w, b):
            pltpu.async_copy(
                h_hbm.at[col_v.at[pl.ds(w * _WIN, _WIN)]], gbuf.at[b],
                gsem[b])

        def drain_gather(b):
            pltpu.make_async_copy(
                h_hbm.at[pl.ds(0, _WIN)], gbuf.at[b], gsem[b]).wait()

        def scale_half(w, half, b, t):
            gb = gbuf.at[b]
            sb = sbuf.at[t]
            base = half * _HWIN

            @plsc.parallel_loop(0, _HWIN, step=2, unroll=4)
            def _edge(e):
                for u in range(2):
                    idx16 = jnp.full((_LANES,),
                                     w * _WIN + base + e + u, jnp.int32)
                    vs = plsc.load_gather(ev_v, [idx16])
                    words = [gb[base + e + u, pl.ds(j, _LANES)]
                             for j in range(0, _D // 2, _LANES)]
                    for k, word in enumerate(words):
                        lo = plsc.bitcast(word << 16, jnp.float32)
                        hi = plsc.bitcast(word & jnp.int32(-65536),
                                          jnp.float32)
                        sb[e + u, pl.ds(2 * k * _LANES, _LANES)] = lo * vs
                        sb[e + u, pl.ds((2 * k + 1) * _LANES,
                                        _LANES)] = hi * vs

        def start_scatter(sw, t):
            pltpu.async_copy(sbuf.at[t], acc.at[row_v.at[sw]], ssem[t],
                             add=True)

        def drain_scatter(t):
            pltpu.make_async_copy(
                out_hbm.at[0, pl.ds(0, _HWIN)], sbuf.at[t], ssem[t]).wait()

        for ph in range(_NPH):
            pltpu.sync_copy(col_hbm.at[wid * _NPH + ph], col_v)
            pltpu.sync_copy(row_hbm.at[wid * _NPH + ph], row_v)
            pltpu.sync_copy(ev_hbm.at[wid * _NPH + ph], ev_v)
            for b in range(_NBUF):
                start_gather(b, b)

            @pl.loop(0, _WPH, step=_NBUF)
            def _window(w):
                for b in range(_NBUF):
                    drain_gather(b)
                    for half in range(2):
                        p = 2 * b + half
                        t = p % _NBUF
                        if p < _NBUF:
                            @pl.when(w > 0)
                            def _():
                                drain_scatter(t)
                        else:
                            drain_scatter(t)
                        scale_half(w + b, half, b, t)
                        start_scatter(2 * (w + b) + half, t)

                    @pl.when(w + _NBUF + b < _WPH)
                    def _():
                        start_gather(w + _NBUF + b, b)

            for t in range(_NBUF):
                drain_scatter(t)

        plsc.subcore_barrier()

        @pl.loop(0, pl.cdiv(_NCHUNK, _NS))
        def _flush(k):
            chunk = s + k * _NS

            @pl.when(chunk < _NCHUNK)
            def _():
                r0 = chunk * _CHUNK
                pltpu.sync_copy(acc.at[pl.ds(r0, _CHUNK)],
                                out_hbm.at[c, pl.ds(r0, _CHUNK)])

    return spmm(h, row3, col2, ev2)


def kernel(x, edge_index, edge_values, W1, W2):
    pad2 = ((0, 0), (0, _PAD))
    row3 = jnp.pad(edge_index[0].reshape(_NW, _EPW), pad2).reshape(
        _NW * _NPH, 2 * _WPH, _HWIN)
    col2 = jnp.pad(edge_index[1].reshape(_NW, _EPW), pad2).reshape(
        _NW * _NPH, _EPH)
    ev2 = jnp.pad(edge_values.reshape(_NW, _EPW), pad2).reshape(
        _NW * _NPH, _EPH)

    perm = jnp.array(_PERM, dtype=jnp.int32)
    h1 = _tc_matmul(x, W1[perm])
    p = _sc_spmm(_pack_i32(h1), row3, col2, ev2)
    h2 = _tc_relu_add_matmul(p, W2[perm])
    q = _sc_spmm(_pack_i32(h2), row3, col2, ev2)
    return _tc_add(q)


# speedup vs baseline: 1.0041x; 1.0041x over previous
import dataclasses
import functools

import jax
import jax.numpy as jnp
from jax import lax
from jax.experimental import pallas as pl
from jax.experimental.pallas import tpu as pltpu
from jax.experimental.pallas import tpu_sc as plsc

_N = 10000
_D = 128
_E = 320000
_NC = 2
_NS = 16
_NW = _NC * _NS
_EPW = _E // _NW
_WIN = 80
_EPP = 10080
_PAD = _EPP - _EPW
_NPH = 3
_EPH = _EPP // _NPH
_WPH = _EPH // _WIN
_NBUF = 3
_HWIN = _WIN // 2

_PERM = []
for _j in range(0, _D, 32):
    for _t in range(16):
        _PERM.append(_j + _t)
        _PERM.append(_j + 16 + _t)
_CHUNK = 200
_NCHUNK = _N // _CHUNK
_NZCHUNK = _N // _HWIN
_LANES = 16

_BM = 400


def _pack_i32(h):
    return lax.bitcast_convert_type(
        h.reshape(h.shape[0], _D // 2, 2), jnp.int32)


def _mm_body(x_ref, w_ref, o_ref):
    o_ref[...] = lax.dot_general(
        x_ref[...], w_ref[...], (((1,), (1,)), ((), ())),
        precision=lax.Precision.HIGHEST,
        preferred_element_type=jnp.float32).astype(jnp.bfloat16)


def _tc_matmul(x, w):
    return pl.pallas_call(
        _mm_body,
        grid=(_N // _BM,),
        in_specs=[pl.BlockSpec((_BM, _D), lambda i: (i, 0)),
                  pl.BlockSpec((_D, _D), lambda i: (0, 0))],
        out_specs=pl.BlockSpec((_BM, _D), lambda i: (i, 0)),
        out_shape=jax.ShapeDtypeStruct((_N, _D), jnp.bfloat16),
    )(x, w)


def _fuse_body(p_ref, w_ref, o_ref):
    h = jnp.maximum(p_ref[0] + p_ref[1], 0.0)
    o_ref[...] = lax.dot_general(
        h, w_ref[...], (((1,), (1,)), ((), ())),
        precision=lax.Precision.HIGHEST,
        preferred_element_type=jnp.float32).astype(jnp.bfloat16)


def _tc_relu_add_matmul(p, w):
    return pl.pallas_call(
        _fuse_body,
        grid=(_N // _BM,),
        in_specs=[pl.BlockSpec((_NC, _BM, _D), lambda i: (0, i, 0)),
                  pl.BlockSpec((_D, _D), lambda i: (0, 0))],
        out_specs=pl.BlockSpec((_BM, _D), lambda i: (i, 0)),
        out_shape=jax.ShapeDtypeStruct((_N, _D), jnp.bfloat16),
    )(p, w)


def _add_body(q_ref, o_ref):
    o_ref[...] = q_ref[0] + q_ref[1]


def _tc_add(q):
    return pl.pallas_call(
        _add_body,
        grid=(_N // _BM,),
        in_specs=[pl.BlockSpec((_NC, _BM, _D), lambda i: (0, i, 0))],
        out_specs=pl.BlockSpec((_BM, _D), lambda i: (i, 0)),
        out_shape=jax.ShapeDtypeStruct((_N, _D), jnp.float32),
    )(q)


def _sc_spmm(h, row3, col2, ev2):
    mesh = plsc.VectorSubcoreMesh(core_axis_name="c", subcore_axis_name="s")
    cp = pltpu.CompilerParams()
    if "needs_layout_passes" in pltpu.CompilerParams.__dataclass_fields__:
        cp = dataclasses.replace(cp, needs_layout_passes=False)
    if "use_tc_tiling_on_sc" in pltpu.CompilerParams.__dataclass_fields__:
        cp = dataclasses.replace(cp, use_tc_tiling_on_sc=False)

    @functools.partial(
        pl.kernel,
        out_type=jax.ShapeDtypeStruct((_NC, _N, _D), jnp.float32),
        mesh=mesh,
        compiler_params=cp,
        scratch_types=[
            pltpu.VMEM((_EPH,), jnp.int32),
            pltpu.VMEM((2 * _WPH, _HWIN), jnp.int32),
            pltpu.VMEM((_EPH,), jnp.float32),
            pltpu.VMEM((_NBUF, _WIN, _D // 2), jnp.int32),
            pltpu.VMEM((_NBUF, _HWIN, _D), jnp.float32),
            pltpu.VMEM_SHARED((_N, _D), jnp.float32),
            pltpu.SemaphoreType.DMA,
            pltpu.SemaphoreType.DMA,
            pltpu.SemaphoreType.DMA,
            pltpu.SemaphoreType.DMA,
            pltpu.SemaphoreType.DMA,
            pltpu.SemaphoreType.DMA,
        ],
    )
    def spmm(h_hbm, row_hbm, col_hbm, ev_hbm, out_hbm,
             col_v, row_v, ev_v, gbuf, sbuf, acc,
             gs0, gs1, gs2, ss0, ss1, ss2):
        c = lax.axis_index("c")
        s = lax.axis_index("s")
        wid = s * _NC + c
        gsem = (gs0, gs1, gs2)
        ssem = (ss0, ss1, ss2)

        zb = sbuf.at[0]

        @pl.loop(0, _HWIN)
        def _zero_stage(i):
            for j in range(0, _D, _LANES):
                zb[i, pl.ds(j, _LANES)] = jnp.zeros((_LANES,), jnp.float32)

        @pl.loop(0, pl.cdiv(_NZCHUNK, _NS))
        def _zero_acc(k):
            chunk = s + k * _NS

            @pl.when(chunk < _NZCHUNK)
            def _():
                pltpu.sync_copy(zb, acc.at[pl.ds(chunk * _HWIN, _HWIN)])

        plsc.subcore_barrier()

        def start_gather(w, b):
            pltpu.async_copy(
                h_hbm.at[col_v.at[pl.ds(w * _WIN, _WIN)]], gbuf.at[b],
                gsem[b])

        def drain_gather(b):
            pltpu.make_async_copy(
                h_hbm.at[pl.ds(0, _WIN)], gbuf.at[b], gsem[b]).wait()

        def scale_half(w, half, b, t):
            gb = gbuf.at[b]
            sb = sbuf.at[t]
            base = half * _HWIN

            @plsc.parallel_loop(0, _HWIN, step=2, unroll=2)
            def _edge(e):
                for u in range(2):
                    idx16 = jnp.full((_LANES,),
                                     w * _WIN + base + e + u, jnp.int32)
                    vs = plsc.load_gather(ev_v, [idx16])
                    words = [gb[base + e + u, pl.ds(j, _LANES)]
                             for j in range(0, _D // 2, _LANES)]
                    for k, word in enumerate(words):
                        lo = plsc.bitcast(word << 16, jnp.float32)
                        hi = plsc.bitcast(word & jnp.int32(-65536),
                                          jnp.float32)
                        sb[e + u, pl.ds(2 * k * _LANES, _LANES)] = lo * vs
                        sb[e + u, pl.ds((2 * k + 1) * _LANES,
                                        _LANES)] = hi * vs

        def start_scatter(sw, t):
            pltpu.async_copy(sbuf.at[t], acc.at[row_v.at[sw]], ssem[t],
                             add=True)

        def drain_scatter(t):
            pltpu.make_async_copy(
                out_hbm.at[0, pl.ds(0, _HWIN)], sbuf.at[t], ssem[t]).wait()

        for ph in range(_NPH):
            pltpu.sync_copy(col_hbm.at[wid * _NPH + ph], col_v)
            pltpu.sync_copy(row_hbm.at[wid * _NPH + ph], row_v)
            pltpu.sync_copy(ev_hbm.at[wid * _NPH + ph], ev_v)
            for b in range(_NBUF):
                start_gather(b, b)

            @pl.loop(0, _WPH, step=_NBUF)
            def _window(w):
                for b in range(_NBUF):
                    drain_gather(b)
                    for half in range(2):
                        p = 2 * b + half
                        t = p % _NBUF
                        if p < _NBUF:
                            @pl.when(w > 0)
                            def _():
                                drain_scatter(t)
                        else:
                            drain_scatter(t)
                        scale_half(w + b, half, b, t)
                        start_scatter(2 * (w + b) + half, t)

                    @pl.when(w + _NBUF + b < _WPH)
                    def _():
                        start_gather(w + _NBUF + b, b)

            for t in range(_NBUF):
                drain_scatter(t)

        plsc.subcore_barrier()

        @pl.loop(0, pl.cdiv(_NCHUNK, _NS))
        def _flush(k):
            chunk = s + k * _NS

            @pl.when(chunk < _NCHUNK)
            def _():
                r0 = chunk * _CHUNK
                pltpu.sync_copy(acc.at[pl.ds(r0, _CHUNK)],
                                out_hbm.at[c, pl.ds(r0, _CHUNK)])

    return spmm(h, row3, col2, ev2)


def kernel(x, edge_index, edge_values, W1, W2):
    pad2 = ((0, 0), (0, _PAD))
    row3 = jnp.pad(edge_index[0].reshape(_NW, _EPW), pad2).reshape(
        _NW * _NPH, 2 * _WPH, _HWIN)
    col2 = jnp.pad(edge_index[1].reshape(_NW, _EPW), pad2).reshape(
        _NW * _NPH, _EPH)
    ev2 = jnp.pad(edge_values.reshape(_NW, _EPW), pad2).reshape(
        _NW * _NPH, _EPH)

    perm = jnp.array(_PERM, dtype=jnp.int32)
    h1 = _tc_matmul(x, W1[perm])
    p = _sc_spmm(_pack_i32(h1), row3, col2, ev2)
    h2 = _tc_relu_add_matmul(p, W2[perm])
    q = _sc_spmm(_pack_i32(h2), row3, col2, ev2)
    return _tc_add(q)
